# baseline (speedup 1.0000x reference)
import functools

import jax
import jax.numpy as jnp
from jax import lax
from jax.experimental import pallas as pl
from jax.experimental.pallas import tpu as pltpu
from jax.experimental.pallas import tpu_sc as plsc

N = 10000
E = 320000
T = 12
T2 = T // 2
F_IN = 128
F_OUT = 32
C2 = 2 * F_OUT
C4 = 2 * C2

NC, NS, L = 2, 16, 16
NW = NC * NS
K = 128
CH = 80
CHA = 32
CHB = 128
NQ = 4
CHA4 = CHA // NQ
CHB4 = CHB // NQ
TOTCH = NS * (CHA + CHB)
EPW = CH * K
E_PAD = NW * EPW
ACC_ROWS = 10240
RPS = ACC_ROWS // NS
JUNK = N + 64
ZR = 32

_mesh = plsc.VectorSubcoreMesh(core_axis_name="c", subcore_axis_name="s")


def _deg_body(dst_ref, out_ref, dstv, onesv, zerov, acc):
    c = lax.axis_index("c")
    s = lax.axis_index("s")
    ones16 = jnp.ones((L,), jnp.float32)
    for i in range(K // L):
        onesv[pl.ds(i * L, L)] = ones16

    def zfill(r, _):
        zerov[pl.ds(r * L, L)] = jnp.zeros((L,), jnp.float32)
        return 0

    lax.fori_loop(0, RPS // L, zfill, 0)

    pltpu.sync_copy(dst_ref.at[c, s], dstv)
    pltpu.sync_copy(zerov, acc.at[pl.ds(s * RPS, RPS)])
    plsc.subcore_barrier()

    def chunk(j, _):
        pltpu.sync_copy(onesv, acc.at[dstv.at[j]], add=True)
        return 0

    lax.fori_loop(0, CH, chunk, 0)
    plsc.subcore_barrier()
    pltpu.sync_copy(acc.at[pl.ds(s * RPS, RPS)], out_ref.at[c, pl.ds(s * RPS, RPS)])


_deg_call = pl.kernel(
    _deg_body,
    mesh=_mesh,
    out_type=jax.ShapeDtypeStruct((NC, ACC_ROWS), jnp.float32),
    scratch_types=[
        pltpu.VMEM((CH, K), jnp.int32),
        pltpu.VMEM((K,), jnp.float32),
        pltpu.VMEM((RPS,), jnp.float32),
        pltpu.VMEM_SHARED((ACC_ROWS,), jnp.float32),
    ],
)


def _agg_body(yflat_ref, src_ref, dst_ref, out_ref, srcv, dstv, idxv,
              rows0, rows1, zerov, acc, sg0, sg1, ss0, ss1):
    c = lax.axis_index("c")
    s = lax.axis_index("s")
    rows = (rows0, rows1)
    sg = (sg0, sg1)
    ss = (ss0, ss1)

    def zrow(r, _):
        for i in range(C4 // L):
            zerov[r, pl.ds(i * L, L)] = jnp.zeros((L,), jnp.float32)
        return 0

    lax.fori_loop(0, ZR, zrow, 0)

    def prep_idx(b, j, poff):
        for i in range(K // L):
            idxv[b, pl.ds(i * L, L)] = srcv[j, pl.ds(i * L, L)] + poff

    def gather(b):
        pltpu.async_copy(yflat_ref.at[idxv.at[b]], rows[b], sg[b])

    base = jnp.where(c == 0, s * CHA, NS * CHA + s * CHB)
    nq = jnp.where(c == 0, CHA4, CHB4)

    def period(p, _):
        poff = p * N

        def zslice(q, _):
            pltpu.sync_copy(zerov, acc.at[pl.ds(s * RPS + q * ZR, ZR)])
            return 0

        lax.fori_loop(0, RPS // ZR, zslice, 0)
        plsc.subcore_barrier()

        for h in range(NQ):
            qbase = base + h * nq

            @pl.when(c == 0)
            def _():
                pltpu.sync_copy(src_ref.at[pl.ds(qbase, CHA4)],
                                srcv.at[pl.ds(0, CHA4)])
                pltpu.sync_copy(dst_ref.at[pl.ds(qbase, CHA4)],
                                dstv.at[pl.ds(0, CHA4)])

            @pl.when(c == 1)
            def _():
                pltpu.sync_copy(src_ref.at[pl.ds(qbase, CHB4)], srcv)
                pltpu.sync_copy(dst_ref.at[pl.ds(qbase, CHB4)], dstv)

            for b in range(2):
                prep_idx(b, b, poff)
                gather(b)

            def chunk(jj, _):
                for b in range(2):
                    j = 2 * jj + b
                    pltpu.make_async_copy(
                        yflat_ref.at[idxv.at[b]], rows[b], sg[b]).wait()
                    pltpu.async_copy(rows[b], acc.at[dstv.at[j]], ss[b],
                                     add=True)

                    @pl.when(j + 2 < nq)
                    def _():
                        prep_idx(b, j + 2, poff)

                    pltpu.make_async_copy(
                        rows[b], acc.at[dstv.at[j]], ss[b]).wait()

                    @pl.when(j + 2 < nq)
                    def _():
                        gather(b)
                return 0

            lax.fori_loop(0, nq // 2, chunk, 0)

        plsc.subcore_barrier()
        pltpu.sync_copy(acc.at[pl.ds(s * RPS, RPS)],
                        out_ref.at[c, p, pl.ds(s * RPS, RPS)])
        return 0

    lax.fori_loop(0, T2, period, 0)


_agg_call = pl.kernel(
    _agg_body,
    mesh=_mesh,
    out_type=jax.ShapeDtypeStruct((NC, T2, ACC_ROWS, C4), jnp.float32),
    scratch_types=[
        pltpu.VMEM((CHB4, K), jnp.int32),
        pltpu.VMEM((CHB4, K), jnp.int32),
        pltpu.VMEM((2, K), jnp.int32),
        pltpu.VMEM((K, C4), jnp.float32),
        pltpu.VMEM((K, C4), jnp.float32),
        pltpu.VMEM((ZR, C4), jnp.float32),
        pltpu.VMEM_SHARED((ACC_ROWS, C4), jnp.float32),
        pltpu.SemaphoreType.DMA,
        pltpu.SemaphoreType.DMA,
        pltpu.SemaphoreType.DMA,
        pltpu.SemaphoreType.DMA,
    ],
)

BN = 200


def _proj_body(x_ref, degp_ref, wz_ref, wlz_ref, wh_ref, wlh_ref, y_ref):
    wc = jnp.concatenate(
        [wz_ref[...] @ wlz_ref[:F_OUT], wh_ref[...] @ wlh_ref[:F_OUT]], axis=1)
    dn = (((2,), (0,)), ((), ()))
    ye = lax.dot_general(x_ref[:, 0], wc, dn,
                         preferred_element_type=jnp.float32)
    yo = lax.dot_general(x_ref[:, 1], wc, dn,
                         preferred_element_type=jnp.float32)
    dinv = lax.rsqrt(degp_ref[0] + degp_ref[1] + 1.0)
    y_ref[...] = jnp.concatenate([ye, yo], axis=2) * dinv[None, :, :]


def _proj_call(xt, degp3, wz, wlz, wh, wlh):
    return pl.pallas_call(
        _proj_body,
        grid=(N // BN,),
        in_specs=[
            pl.BlockSpec((T2, 2, BN, F_IN), lambda i: (0, 0, i, 0)),
            pl.BlockSpec((NC, BN, 1), lambda i: (0, i, 0)),
            pl.BlockSpec((F_IN, F_OUT), lambda i: (0, 0)),
            pl.BlockSpec((2 * F_OUT, F_OUT), lambda i: (0, 0)),
            pl.BlockSpec((F_IN, F_OUT), lambda i: (0, 0)),
            pl.BlockSpec((2 * F_OUT, F_OUT), lambda i: (0, 0)),
        ],
        out_specs=pl.BlockSpec((T2, BN, C4), lambda i: (0, i, 0)),
        out_shape=jax.ShapeDtypeStruct((T2, N, C4), jnp.float32),
    )(xt, degp3, wz, wlz, wh, wlh)


def _final_body(part_ref, y_ref, degp_ref, att_ref, bz_ref, wlz_ref, blz_ref,
                bh_ref, wlh_ref, blh_ref, o_ref):
    att = att_ref[...]
    e = jnp.exp(att - jnp.max(att))
    probs = e / jnp.sum(e)
    pe = probs[:, 0:1]
    po = probs[:, 1:2]
    bz = bz_ref[...] @ wlz_ref[:F_OUT] + blz_ref[...]
    bh = bh_ref[...] @ wlh_ref[:F_OUT] + blh_ref[...]
    dinv = lax.rsqrt(degp_ref[0] + degp_ref[1] + 1.0)
    a = (part_ref[0] + part_ref[1] + y_ref[...]) * dinv[None, :, :]
    ge = (1.0 - jax.nn.sigmoid(a[:, :, 0 * F_OUT:1 * F_OUT] + bz[None])) \
        * jnp.tanh(a[:, :, 1 * F_OUT:2 * F_OUT] + bh[None])
    go = (1.0 - jax.nn.sigmoid(a[:, :, 2 * F_OUT:3 * F_OUT] + bz[None])) \
        * jnp.tanh(a[:, :, 3 * F_OUT:4 * F_OUT] + bh[None])
    o_ref[...] = jnp.sum(ge * pe[:, :, None] + go * po[:, :, None], axis=0)


def _final_call(part, y, degp3, att, bz, wlz, blz, bh, wlh, blh):
    return pl.pallas_call(
        _final_body,
        grid=(N // BN,),
        in_specs=[
            pl.BlockSpec((NC, T2, BN, C4), lambda i: (0, 0, i, 0)),
            pl.BlockSpec((T2, BN, C4), lambda i: (0, i, 0)),
            pl.BlockSpec((NC, BN, 1), lambda i: (0, i, 0)),
            pl.BlockSpec((T2, 2), lambda i: (0, 0)),
            pl.BlockSpec((1, F_OUT), lambda i: (0, 0)),
            pl.BlockSpec((2 * F_OUT, F_OUT), lambda i: (0, 0)),
            pl.BlockSpec((1, F_OUT), lambda i: (0, 0)),
            pl.BlockSpec((1, F_OUT), lambda i: (0, 0)),
            pl.BlockSpec((2 * F_OUT, F_OUT), lambda i: (0, 0)),
            pl.BlockSpec((1, F_OUT), lambda i: (0, 0)),
        ],
        out_specs=pl.BlockSpec((BN, F_OUT), lambda i: (i, 0)),
        out_shape=jax.ShapeDtypeStruct((N, F_OUT), jnp.float32),
    )(part, y, degp3, att, bz, wlz, blz, bh, wlh, blh)


def kernel(X, edge_index, W_z, b_z, Wl_z, bl_z, W_r, b_r, Wl_r, bl_r,
           W_h, b_h, Wl_h, bl_h, attention):
    pad = E_PAD - E
    src_r = jnp.concatenate(
        [edge_index[0], jnp.zeros((pad,), jnp.int32)]).reshape(TOTCH, K)
    dst_r = jnp.concatenate(
        [edge_index[1], jnp.full((pad,), JUNK, jnp.int32)]).reshape(TOTCH, K)

    degp = _deg_call(dst_r.reshape(NC, NS, CH, K))
    degp3 = degp[:, :, None]

    xt = jnp.transpose(X, (2, 0, 1)).reshape(T2, 2, N, F_IN)
    y = _proj_call(xt, degp3, W_z, Wl_z, W_h, Wl_h)

    part = _agg_call(y.reshape(T2 * N, C4), src_r, dst_r)

    return _final_call(
        part, y, degp3, attention.reshape(T2, 2),
        b_z.reshape(1, F_OUT), Wl_z, bl_z.reshape(1, F_OUT),
        b_h.reshape(1, F_OUT), Wl_h, bl_h.reshape(1, F_OUT))


# speedup vs baseline: 1.4233x; 1.4233x over previous
import functools

import jax
import jax.numpy as jnp
from jax import lax
from jax.experimental import pallas as pl
from jax.experimental.pallas import tpu as pltpu
from jax.experimental.pallas import tpu_sc as plsc

N = 10000
E = 320000
T = 12
T2 = T // 2
F_IN = 128
F_OUT = 32
C2 = 2 * F_OUT
C4 = 2 * C2

NC, NS, L = 2, 16, 16
NW = NC * NS
K = 128
CH = 80
P0 = 4
TOTCH = 2560
CHS = TOTCH // NS
NQ = 4
CHQ = CHS // NQ
EPW = CH * K
E_PAD = NW * EPW
ACC_ROWS = 10240
RPS = ACC_ROWS // NS
JUNK = N + 64
ZR = 32

_mesh = plsc.VectorSubcoreMesh(core_axis_name="c", subcore_axis_name="s")


def _deg_body(dst_ref, out_ref, dstv, onesv, zerov, acc):
    c = lax.axis_index("c")
    s = lax.axis_index("s")
    ones16 = jnp.ones((L,), jnp.float32)
    for i in range(K // L):
        onesv[pl.ds(i * L, L)] = ones16

    def zfill(r, _):
        zerov[pl.ds(r * L, L)] = jnp.zeros((L,), jnp.float32)
        return 0

    lax.fori_loop(0, RPS // L, zfill, 0)

    pltpu.sync_copy(dst_ref.at[c, s], dstv)
    pltpu.sync_copy(zerov, acc.at[pl.ds(s * RPS, RPS)])
    plsc.subcore_barrier()

    def chunk(j, _):
        pltpu.sync_copy(onesv, acc.at[dstv.at[j]], add=True)
        return 0

    lax.fori_loop(0, CH, chunk, 0)
    plsc.subcore_barrier()
    pltpu.sync_copy(acc.at[pl.ds(s * RPS, RPS)], out_ref.at[c, pl.ds(s * RPS, RPS)])


_deg_call = pl.kernel(
    _deg_body,
    mesh=_mesh,
    out_type=jax.ShapeDtypeStruct((NC, ACC_ROWS), jnp.float32),
    scratch_types=[
        pltpu.VMEM((CH, K), jnp.int32),
        pltpu.VMEM((K,), jnp.float32),
        pltpu.VMEM((RPS,), jnp.float32),
        pltpu.VMEM_SHARED((ACC_ROWS,), jnp.float32),
    ],
)


def _agg_body(yflat_ref, src_ref, dst_ref, out_ref, srcv, dstv, idxv,
              rows0, rows1, zerov, acc, sg0, sg1, ss0, ss1):
    c = lax.axis_index("c")
    s = lax.axis_index("s")
    rows = (rows0, rows1)
    sg = (sg0, sg1)
    ss = (ss0, ss1)

    def zrow(r, _):
        for i in range(C4 // L):
            zerov[r, pl.ds(i * L, L)] = jnp.zeros((L,), jnp.float32)
        return 0

    lax.fori_loop(0, ZR, zrow, 0)

    def prep_idx(b, j, poff):
        for i in range(K // L):
            idxv[b, pl.ds(i * L, L)] = srcv[j, pl.ds(i * L, L)] + poff

    def gather(b):
        pltpu.async_copy(yflat_ref.at[idxv.at[b]], rows[b], sg[b])

    npairs = jnp.where(c == 0, P0, T2 - P0)
    pbase = jnp.where(c == 0, 0, P0)

    def period(pp, _):
        p = pbase + pp
        poff = p * N

        def zslice(q, _):
            pltpu.sync_copy(zerov, acc.at[pl.ds(s * RPS + q * ZR, ZR)])
            return 0

        lax.fori_loop(0, RPS // ZR, zslice, 0)
        plsc.subcore_barrier()

        for h in range(NQ):
            qbase = s * CHS + h * CHQ
            pltpu.sync_copy(src_ref.at[pl.ds(qbase, CHQ)], srcv)
            pltpu.sync_copy(dst_ref.at[pl.ds(qbase, CHQ)], dstv)

            for b in range(2):
                prep_idx(b, b, poff)
                gather(b)

            def chunk(jj, _):
                for b in range(2):
                    j = 2 * jj + b
                    pltpu.make_async_copy(
                        yflat_ref.at[idxv.at[b]], rows[b], sg[b]).wait()
                    pltpu.async_copy(rows[b], acc.at[dstv.at[j]], ss[b],
                                     add=True)

                    @pl.when(j + 2 < CHQ)
                    def _():
                        prep_idx(b, j + 2, poff)

                    pltpu.make_async_copy(
                        rows[b], acc.at[dstv.at[j]], ss[b]).wait()

                    @pl.when(j + 2 < CHQ)
                    def _():
                        gather(b)
                return 0

            lax.fori_loop(0, CHQ // 2, chunk, 0)

        plsc.subcore_barrier()
        pltpu.sync_copy(acc.at[pl.ds(s * RPS, RPS)],
                        out_ref.at[p, pl.ds(s * RPS, RPS)])
        return 0

    lax.fori_loop(0, npairs, period, 0)


_agg_call = pl.kernel(
    _agg_body,
    mesh=_mesh,
    out_type=jax.ShapeDtypeStruct((T2, ACC_ROWS, C4), jnp.float32),
    scratch_types=[
        pltpu.VMEM((CHQ, K), jnp.int32),
        pltpu.VMEM((CHQ, K), jnp.int32),
        pltpu.VMEM((2, K), jnp.int32),
        pltpu.VMEM((K, C4), jnp.float32),
        pltpu.VMEM((K, C4), jnp.float32),
        pltpu.VMEM((ZR, C4), jnp.float32),
        pltpu.VMEM_SHARED((ACC_ROWS, C4), jnp.float32),
        pltpu.SemaphoreType.DMA,
        pltpu.SemaphoreType.DMA,
        pltpu.SemaphoreType.DMA,
        pltpu.SemaphoreType.DMA,
    ],
)

BN = 200


def _proj_body(x_ref, degp_ref, wz_ref, wlz_ref, wh_ref, wlh_ref, y_ref):
    wc = jnp.concatenate(
        [wz_ref[...] @ wlz_ref[:F_OUT], wh_ref[...] @ wlh_ref[:F_OUT]], axis=1)
    dn = (((2,), (0,)), ((), ()))
    ye = lax.dot_general(x_ref[:, 0], wc, dn,
                         preferred_element_type=jnp.float32)
    yo = lax.dot_general(x_ref[:, 1], wc, dn,
                         preferred_element_type=jnp.float32)
    dinv = lax.rsqrt(degp_ref[0] + degp_ref[1] + 1.0)
    y_ref[...] = jnp.concatenate([ye, yo], axis=2) * dinv[None, :, :]


def _proj_call(xt, degp3, wz, wlz, wh, wlh):
    return pl.pallas_call(
        _proj_body,
        grid=(N // BN,),
        in_specs=[
            pl.BlockSpec((T2, 2, BN, F_IN), lambda i: (0, 0, i, 0)),
            pl.BlockSpec((NC, BN, 1), lambda i: (0, i, 0)),
            pl.BlockSpec((F_IN, F_OUT), lambda i: (0, 0)),
            pl.BlockSpec((2 * F_OUT, F_OUT), lambda i: (0, 0)),
            pl.BlockSpec((F_IN, F_OUT), lambda i: (0, 0)),
            pl.BlockSpec((2 * F_OUT, F_OUT), lambda i: (0, 0)),
        ],
        out_specs=pl.BlockSpec((T2, BN, C4), lambda i: (0, i, 0)),
        out_shape=jax.ShapeDtypeStruct((T2, N, C4), jnp.float32),
    )(xt, degp3, wz, wlz, wh, wlh)


def _final_body(part_ref, y_ref, degp_ref, att_ref, bz_ref, wlz_ref, blz_ref,
                bh_ref, wlh_ref, blh_ref, o_ref):
    att = att_ref[...]
    e = jnp.exp(att - jnp.max(att))
    probs = e / jnp.sum(e)
    pe = probs[:, 0:1]
    po = probs[:, 1:2]
    bz = bz_ref[...] @ wlz_ref[:F_OUT] + blz_ref[...]
    bh = bh_ref[...] @ wlh_ref[:F_OUT] + blh_ref[...]
    dinv = lax.rsqrt(degp_ref[0] + degp_ref[1] + 1.0)
    a = (part_ref[...] + y_ref[...]) * dinv[None, :, :]
    ge = (1.0 - jax.nn.sigmoid(a[:, :, 0 * F_OUT:1 * F_OUT] + bz[None])) \
        * jnp.tanh(a[:, :, 1 * F_OUT:2 * F_OUT] + bh[None])
    go = (1.0 - jax.nn.sigmoid(a[:, :, 2 * F_OUT:3 * F_OUT] + bz[None])) \
        * jnp.tanh(a[:, :, 3 * F_OUT:4 * F_OUT] + bh[None])
    o_ref[...] = jnp.sum(ge * pe[:, :, None] + go * po[:, :, None], axis=0)


def _final_call(part, y, degp3, att, bz, wlz, blz, bh, wlh, blh):
    return pl.pallas_call(
        _final_body,
        grid=(N // BN,),
        in_specs=[
            pl.BlockSpec((T2, BN, C4), lambda i: (0, i, 0)),
            pl.BlockSpec((T2, BN, C4), lambda i: (0, i, 0)),
            pl.BlockSpec((NC, BN, 1), lambda i: (0, i, 0)),
            pl.BlockSpec((T2, 2), lambda i: (0, 0)),
            pl.BlockSpec((1, F_OUT), lambda i: (0, 0)),
            pl.BlockSpec((2 * F_OUT, F_OUT), lambda i: (0, 0)),
            pl.BlockSpec((1, F_OUT), lambda i: (0, 0)),
            pl.BlockSpec((1, F_OUT), lambda i: (0, 0)),
            pl.BlockSpec((2 * F_OUT, F_OUT), lambda i: (0, 0)),
            pl.BlockSpec((1, F_OUT), lambda i: (0, 0)),
        ],
        out_specs=pl.BlockSpec((BN, F_OUT), lambda i: (i, 0)),
        out_shape=jax.ShapeDtypeStruct((N, F_OUT), jnp.float32),
    )(part, y, degp3, att, bz, wlz, blz, bh, wlh, blh)


def kernel(X, edge_index, W_z, b_z, Wl_z, bl_z, W_r, b_r, Wl_r, bl_r,
           W_h, b_h, Wl_h, bl_h, attention):
    pad = E_PAD - E
    src_r = jnp.concatenate(
        [edge_index[0], jnp.zeros((pad,), jnp.int32)]).reshape(TOTCH, K)
    dst_r = jnp.concatenate(
        [edge_index[1], jnp.full((pad,), JUNK, jnp.int32)]).reshape(TOTCH, K)

    degp = _deg_call(dst_r.reshape(NC, NS, CH, K))
    degp3 = degp[:, :, None]

    xt = jnp.transpose(X, (2, 0, 1)).reshape(T2, 2, N, F_IN)
    y = _proj_call(xt, degp3, W_z, Wl_z, W_h, Wl_h)

    part = _agg_call(y.reshape(T2 * N, C4), src_r, dst_r)

    return _final_call(
        part, y, degp3, attention.reshape(T2, 2),
        b_z.reshape(1, F_OUT), Wl_z, bl_z.reshape(1, F_OUT),
        b_h.reshape(1, F_OUT), Wl_h, bl_h.reshape(1, F_OUT))


# speedup vs baseline: 1.6602x; 1.1665x over previous
import functools

import jax
import jax.numpy as jnp
from jax import lax
from jax.experimental import pallas as pl
from jax.experimental.pallas import tpu as pltpu
from jax.experimental.pallas import tpu_sc as plsc

N = 10000
E = 320000
T = 12
T2 = T // 2
F_IN = 128
F_OUT = 32
C2 = 2 * F_OUT
C4 = 2 * C2

NC, NS, L = 2, 16, 16
NW = NC * NS
K = 128
CH = 80
P0 = 3
TOTCH = 2560
CHS = TOTCH // NS
NQ = 4
CHQ = CHS // NQ
EPW = CH * K
E_PAD = NW * EPW
ACC_ROWS = 10240
RPS = ACC_ROWS // NS
JUNK = N + 64
ZR = 32

_mesh = plsc.VectorSubcoreMesh(core_axis_name="c", subcore_axis_name="s")


def _deg_body(dst_ref, out_ref, dstv, onesv, zerov, acc):
    c = lax.axis_index("c")
    s = lax.axis_index("s")
    ones16 = jnp.ones((L,), jnp.float32)
    for i in range(K // L):
        onesv[pl.ds(i * L, L)] = ones16

    def zfill(r, _):
        zerov[pl.ds(r * L, L)] = jnp.zeros((L,), jnp.float32)
        return 0

    lax.fori_loop(0, RPS // L, zfill, 0)

    pltpu.sync_copy(dst_ref.at[c, s], dstv)
    pltpu.sync_copy(zerov, acc.at[pl.ds(s * RPS, RPS)])
    plsc.subcore_barrier()

    def chunk(j, _):
        pltpu.sync_copy(onesv, acc.at[dstv.at[j]], add=True)
        return 0

    lax.fori_loop(0, CH, chunk, 0)
    plsc.subcore_barrier()
    pltpu.sync_copy(acc.at[pl.ds(s * RPS, RPS)], out_ref.at[c, pl.ds(s * RPS, RPS)])


_deg_call = pl.kernel(
    _deg_body,
    mesh=_mesh,
    out_type=jax.ShapeDtypeStruct((NC, ACC_ROWS), jnp.float32),
    scratch_types=[
        pltpu.VMEM((CH, K), jnp.int32),
        pltpu.VMEM((K,), jnp.float32),
        pltpu.VMEM((RPS,), jnp.float32),
        pltpu.VMEM_SHARED((ACC_ROWS,), jnp.float32),
    ],
)


def _agg_body(yflat_ref, src_ref, dst_ref, out_ref, srcv, dstv, idxv,
              rows0, rows1, zerov, acc, sg0, sg1, ss0, ss1):
    c = lax.axis_index("c")
    s = lax.axis_index("s")
    rows = (rows0, rows1)
    sg = (sg0, sg1)
    ss = (ss0, ss1)

    def zrow(r, _):
        for i in range(C4 // L):
            zerov[r, pl.ds(i * L, L)] = jnp.zeros((L,), jnp.float32)
        return 0

    lax.fori_loop(0, ZR, zrow, 0)

    def prep_idx(b, j, poff):
        for i in range(K // L):
            idxv[b, pl.ds(i * L, L)] = srcv[j, pl.ds(i * L, L)] + poff

    def gather(b):
        pltpu.async_copy(yflat_ref.at[idxv.at[b]], rows[b], sg[b])

    npairs = jnp.where(c == 0, P0, T2 - P0)
    pbase = jnp.where(c == 0, 0, P0)

    def period(pp, _):
        p = pbase + pp
        poff = p * N

        def zslice(q, _):
            pltpu.sync_copy(zerov, acc.at[pl.ds(s * RPS + q * ZR, ZR)])
            return 0

        lax.fori_loop(0, RPS // ZR, zslice, 0)
        plsc.subcore_barrier()

        for h in range(NQ):
            qbase = s * CHS + h * CHQ
            pltpu.sync_copy(src_ref.at[pl.ds(qbase, CHQ)], srcv)
            pltpu.sync_copy(dst_ref.at[pl.ds(qbase, CHQ)], dstv)

            for b in range(2):
                prep_idx(b, b, poff)
                gather(b)

            def chunk(jj, _):
                for b in range(2):
                    j = 2 * jj + b
                    pltpu.make_async_copy(
                        yflat_ref.at[idxv.at[b]], rows[b], sg[b]).wait()
                    pltpu.async_copy(rows[b], acc.at[dstv.at[j]], ss[b],
                                     add=True)

                    @pl.when(j + 2 < CHQ)
                    def _():
                        prep_idx(b, j + 2, poff)

                    pltpu.make_async_copy(
                        rows[b], acc.at[dstv.at[j]], ss[b]).wait()

                    @pl.when(j + 2 < CHQ)
                    def _():
                        gather(b)
                return 0

            lax.fori_loop(0, CHQ // 2, chunk, 0)

        plsc.subcore_barrier()
        pltpu.sync_copy(acc.at[pl.ds(s * RPS, RPS)],
                        out_ref.at[p, pl.ds(s * RPS, RPS)])
        return 0

    lax.fori_loop(0, npairs, period, 0)


_agg_call = pl.kernel(
    _agg_body,
    mesh=_mesh,
    out_type=jax.ShapeDtypeStruct((T2, ACC_ROWS, C4), jnp.float32),
    scratch_types=[
        pltpu.VMEM((CHQ, K), jnp.int32),
        pltpu.VMEM((CHQ, K), jnp.int32),
        pltpu.VMEM((2, K), jnp.int32),
        pltpu.VMEM((K, C4), jnp.float32),
        pltpu.VMEM((K, C4), jnp.float32),
        pltpu.VMEM((ZR, C4), jnp.float32),
        pltpu.VMEM_SHARED((ACC_ROWS, C4), jnp.float32),
        pltpu.SemaphoreType.DMA,
        pltpu.SemaphoreType.DMA,
        pltpu.SemaphoreType.DMA,
        pltpu.SemaphoreType.DMA,
    ],
)

BN = 200


def _proj_body(x_ref, degp_ref, wz_ref, wlz_ref, wh_ref, wlh_ref, y_ref):
    wc = jnp.concatenate(
        [wz_ref[...] @ wlz_ref[:F_OUT], wh_ref[...] @ wlh_ref[:F_OUT]], axis=1)
    dn = (((2,), (0,)), ((), ()))
    ye = lax.dot_general(x_ref[:, 0], wc, dn,
                         preferred_element_type=jnp.float32)
    yo = lax.dot_general(x_ref[:, 1], wc, dn,
                         preferred_element_type=jnp.float32)
    dinv = lax.rsqrt(degp_ref[0] + degp_ref[1] + 1.0)
    y_ref[...] = jnp.concatenate([ye, yo], axis=2) * dinv[None, :, :]


def _proj_call(xt, degp3, wz, wlz, wh, wlh):
    return pl.pallas_call(
        _proj_body,
        grid=(N // BN,),
        in_specs=[
            pl.BlockSpec((T2, 2, BN, F_IN), lambda i: (0, 0, i, 0)),
            pl.BlockSpec((NC, BN, 1), lambda i: (0, i, 0)),
            pl.BlockSpec((F_IN, F_OUT), lambda i: (0, 0)),
            pl.BlockSpec((2 * F_OUT, F_OUT), lambda i: (0, 0)),
            pl.BlockSpec((F_IN, F_OUT), lambda i: (0, 0)),
            pl.BlockSpec((2 * F_OUT, F_OUT), lambda i: (0, 0)),
        ],
        out_specs=pl.BlockSpec((T2, BN, C4), lambda i: (0, i, 0)),
        out_shape=jax.ShapeDtypeStruct((T2, N, C4), jnp.float32),
    )(xt, degp3, wz, wlz, wh, wlh)


def _final_body(part_ref, y_ref, degp_ref, att_ref, bz_ref, wlz_ref, blz_ref,
                bh_ref, wlh_ref, blh_ref, o_ref):
    att = att_ref[...]
    e = jnp.exp(att - jnp.max(att))
    probs = e / jnp.sum(e)
    pe = probs[:, 0:1]
    po = probs[:, 1:2]
    bz = bz_ref[...] @ wlz_ref[:F_OUT] + blz_ref[...]
    bh = bh_ref[...] @ wlh_ref[:F_OUT] + blh_ref[...]
    dinv = lax.rsqrt(degp_ref[0] + degp_ref[1] + 1.0)
    a = (part_ref[...] + y_ref[...]) * dinv[None, :, :]
    ge = (1.0 - jax.nn.sigmoid(a[:, :, 0 * F_OUT:1 * F_OUT] + bz[None])) \
        * jnp.tanh(a[:, :, 1 * F_OUT:2 * F_OUT] + bh[None])
    go = (1.0 - jax.nn.sigmoid(a[:, :, 2 * F_OUT:3 * F_OUT] + bz[None])) \
        * jnp.tanh(a[:, :, 3 * F_OUT:4 * F_OUT] + bh[None])
    o_ref[...] = jnp.sum(ge * pe[:, :, None] + go * po[:, :, None], axis=0)


def _final_call(part, y, degp3, att, bz, wlz, blz, bh, wlh, blh):
    return pl.pallas_call(
        _final_body,
        grid=(N // BN,),
        in_specs=[
            pl.BlockSpec((T2, BN, C4), lambda i: (0, i, 0)),
            pl.BlockSpec((T2, BN, C4), lambda i: (0, i, 0)),
            pl.BlockSpec((NC, BN, 1), lambda i: (0, i, 0)),
            pl.BlockSpec((T2, 2), lambda i: (0, 0)),
            pl.BlockSpec((1, F_OUT), lambda i: (0, 0)),
            pl.BlockSpec((2 * F_OUT, F_OUT), lambda i: (0, 0)),
            pl.BlockSpec((1, F_OUT), lambda i: (0, 0)),
            pl.BlockSpec((1, F_OUT), lambda i: (0, 0)),
            pl.BlockSpec((2 * F_OUT, F_OUT), lambda i: (0, 0)),
            pl.BlockSpec((1, F_OUT), lambda i: (0, 0)),
        ],
        out_specs=pl.BlockSpec((BN, F_OUT), lambda i: (i, 0)),
        out_shape=jax.ShapeDtypeStruct((N, F_OUT), jnp.float32),
    )(part, y, degp3, att, bz, wlz, blz, bh, wlh, blh)


def kernel(X, edge_index, W_z, b_z, Wl_z, bl_z, W_r, b_r, Wl_r, bl_r,
           W_h, b_h, Wl_h, bl_h, attention):
    pad = E_PAD - E
    src_r = jnp.concatenate(
        [edge_index[0], jnp.zeros((pad,), jnp.int32)]).reshape(TOTCH, K)
    dst_r = jnp.concatenate(
        [edge_index[1], jnp.full((pad,), JUNK, jnp.int32)]).reshape(TOTCH, K)

    degp = _deg_call(dst_r.reshape(NC, NS, CH, K))
    degp3 = degp[:, :, None]

    xt = jnp.transpose(X, (2, 0, 1)).reshape(T2, 2, N, F_IN)
    y = _proj_call(xt, degp3, W_z, Wl_z, W_h, Wl_h)

    part = _agg_call(y.reshape(T2 * N, C4), src_r, dst_r)

    return _final_call(
        part, y, degp3, attention.reshape(T2, 2),
        b_z.reshape(1, F_OUT), Wl_z, bl_z.reshape(1, F_OUT),
        b_h.reshape(1, F_OUT), Wl_h, bl_h.reshape(1, F_OUT))
